# Initial kernel scaffold; baseline (speedup 1.0000x reference)
import jax
import jax.numpy as jnp
from jax.experimental import pallas as pl


def kernel(outputs):
    raise NotImplementedError("write your pallas kernel here")


# speedup vs baseline: 1.1399x; 1.1399x over previous
import functools

import jax
import jax.numpy as jnp
import numpy as np
from jax import lax
from jax.experimental import pallas as pl
from jax.experimental.pallas import tpu as pltpu
from jax.experimental.pallas import tpu_sc as plsc

_C = 1000
_DEPTH = 10
_PAD = [max(16, 1 << d) for d in range(_DEPTH + 1)]
_OFFS = [0] + [int(x) for x in np.cumsum(_PAD)]
_SZ = int(_OFFS[_DEPTH + 1])
_OUT_PAD = 1008


def _build_tables():
    g10 = np.zeros(_PAD[_DEPTH], np.int32)
    w10 = np.zeros(_PAD[_DEPTH], np.float32)
    eidx = {d: np.zeros(_PAD[d], np.int32) for d in range(1, _DEPTH)}
    oidx = {d: np.zeros(_PAD[d], np.int32) for d in range(1, _DEPTH)}
    sib = {d: np.zeros(_PAD[d], np.int32) for d in range(1, _DEPTH + 1)}
    ics = {d: np.ones(_PAD[d], np.float32) for d in range(1, _DEPTH + 1)}
    icb = {d: np.ones(_PAD[d], np.float32) for d in range(1, _DEPTH + 1)}
    pa = {d: np.ones(_PAD[d], np.float32) for d in range(1, _DEPTH + 1)}
    pb = {d: np.zeros(_PAD[d], np.float32) for d in range(1, _DEPTH + 1)}
    par = {d: np.zeros(_PAD[d], np.int32) for d in range(2, _DEPTH + 1)}
    slot_out = np.zeros(_OUT_PAD, np.int32)
    counts = {d: np.zeros(_PAD[d], np.float64) for d in range(1, _DEPTH + 1)}

    def rec(a, b, d, p):
        counts[d][p] = b - a
        sib[d][p] = _OFFS[d] + (p ^ 1)
        if d >= 2:
            par[d][p] = _OFFS[d - 1] + p // 2
        if b - a == 1:
            q = p
            for dd in range(d, _DEPTH):
                eidx[dd][q] = _OFFS[dd + 1] + 2 * q
                oidx[dd][q] = 0
                qn = 2 * q
                counts[dd + 1][qn] = 1
                sib[dd + 1][qn] = _OFFS[dd + 1] + (qn ^ 1)
                par[dd + 1][qn] = _OFFS[dd] + q
                pa[dd + 1][qn] = 0.0
                pb[dd + 1][qn] = 1.0
                q = qn
            g10[q] = a
            w10[q] = 1.0
            slot_out[a] = _OFFS[_DEPTH] + q
        else:
            mid = a + (b - a) // 2
            eidx[d][p] = _OFFS[d + 1] + 2 * p
            oidx[d][p] = _OFFS[d + 1] + 2 * p + 1
            rec(a, mid, d + 1, 2 * p)
            rec(mid, b, d + 1, 2 * p + 1)

    mid = _C // 2
    rec(0, mid, 1, 0)
    rec(mid, _C, 1, 1)

    for d in range(1, _DEPTH + 1):
        nz = counts[d] > 0
        ics[d][nz] = (1.0 / counts[d][nz]).astype(np.float32)
        cs = counts[d][np.arange(_PAD[d]) ^ 1]
        nz2 = nz & (cs > 0)
        icb[d][nz2] = (1.0 / cs[nz2]).astype(np.float32)

    ioffs = {}
    iparts = []

    def iadd(name, arr):
        ioffs[name] = sum(len(x) for x in iparts)
        iparts.append(arr.astype(np.int32))

    foffs = {}
    fparts = []

    def fadd(name, arr):
        foffs[name] = sum(len(x) for x in fparts)
        fparts.append(arr.astype(np.float32))

    iadd("g10", g10)
    for d in range(1, _DEPTH):
        iadd(("e", d), eidx[d])
        iadd(("o", d), oidx[d])
    for d in range(1, _DEPTH + 1):
        iadd(("sib", d), sib[d])
    for d in range(2, _DEPTH + 1):
        iadd(("par", d), par[d])
    iadd("slot", slot_out)

    fadd("w10", w10)
    for d in range(1, _DEPTH + 1):
        fadd(("ics", d), ics[d])
        fadd(("icb", d), icb[d])
        fadd(("pa", d), pa[d])
        fadd(("pb", d), pb[d])

    itab = np.concatenate(iparts)
    ftab = np.concatenate(fparts)
    w10_need = [bool(np.any(w10[16 * k:16 * k + 16] != 1.0)) for k in range(_PAD[_DEPTH] // 16)]
    pa_need = {
        d: [bool(np.any(pa[d][16 * k:16 * k + 16] != 1.0)) for k in range(_PAD[d] // 16)]
        for d in range(1, _DEPTH + 1)
    }
    return itab, ftab, ioffs, foffs, w10_need, pa_need


_ITAB, _FTAB, _IOFFS, _FOFFS, _W10_NEED, _PA_NEED = _build_tables()

_INFO = plsc.get_sparse_core_info()
_NW = _INFO.num_cores * _INFO.num_subcores


def _row_program(row_v, sums_v, prob_v, orow_v, itab_v, ftab_v):
    def li(name, k):
        return itab_v[pl.ds(_IOFFS[name] + 16 * k, 16)]

    def lf(name, k):
        return ftab_v[pl.ds(_FOFFS[name] + 16 * k, 16)]

    for k in range(_PAD[_DEPTH] // 16):
        v = plsc.load_gather(row_v, [li("g10", k)])
        if _W10_NEED[k]:
            v = v * lf("w10", k)
        sums_v[pl.ds(_OFFS[_DEPTH] + 16 * k, 16)] = v
    for d in range(_DEPTH - 1, 0, -1):
        for k in range(_PAD[d] // 16):
            a = plsc.load_gather(sums_v, [li(("e", d), k)])
            b = plsc.load_gather(sums_v, [li(("o", d), k)])
            sums_v[pl.ds(_OFFS[d] + 16 * k, 16)] = a + b
    for d in range(1, _DEPTH + 1):
        for k in range(_PAD[d] // 16):
            s = sums_v[pl.ds(_OFFS[d] + 16 * k, 16)]
            sv = plsc.load_gather(sums_v, [li(("sib", d), k)])
            x = sv * lf(("icb", d), k) - s * lf(("ics", d), k)
            p = 1.0 / (1.0 + jnp.exp(x))
            if _PA_NEED[d][k]:
                p = p * lf(("pa", d), k) + lf(("pb", d), k)
            prob_v[pl.ds(_OFFS[d] + 16 * k, 16)] = p
    for d in range(2, _DEPTH + 1):
        for k in range(_PAD[d] // 16):
            pv = plsc.load_gather(prob_v, [li(("par", d), k)])
            cur = prob_v[pl.ds(_OFFS[d] + 16 * k, 16)]
            prob_v[pl.ds(_OFFS[d] + 16 * k, 16)] = cur * pv
    for k in range(_OUT_PAD // 16):
        orow_v[pl.ds(16 * k, 16)] = plsc.load_gather(prob_v, [li("slot", k)])


def _sc_body(rows_per_w, x_hbm, itab_hbm, ftab_hbm, out_hbm,
             itab_v, ftab_v, row_v, sums_v, prob_v, orow_v):
    wid = lax.axis_index("s") * _INFO.num_cores + lax.axis_index("c")
    pltpu.sync_copy(itab_hbm, itab_v)
    pltpu.sync_copy(ftab_hbm, ftab_v)
    zero = jnp.zeros((16,), jnp.float32)
    sums_v[pl.ds(0, 16)] = zero
    prob_v[pl.ds(0, 16)] = zero
    base = wid * rows_per_w

    def body(i, carry):
        r = base + i
        pltpu.sync_copy(x_hbm.at[pl.ds(r * _C, _C)], row_v)
        _row_program(row_v, sums_v, prob_v, orow_v, itab_v, ftab_v)
        pltpu.sync_copy(orow_v.at[pl.ds(0, _C)], out_hbm.at[pl.ds(r * _C, _C)])
        return carry

    lax.fori_loop(0, rows_per_w, body, 0)


@jax.jit
def kernel(outputs):
    B = outputs.shape[0]
    assert B % _NW == 0
    rows_per_w = B // _NW
    mesh = plsc.VectorSubcoreMesh(core_axis_name="c", subcore_axis_name="s")
    fn = pl.kernel(
        functools.partial(_sc_body, rows_per_w),
        out_type=jax.ShapeDtypeStruct((B * _C,), jnp.float32),
        mesh=mesh,
        scratch_types=[
            pltpu.VMEM((len(_ITAB),), jnp.int32),
            pltpu.VMEM((len(_FTAB),), jnp.float32),
            pltpu.VMEM((_C,), jnp.float32),
            pltpu.VMEM((_SZ,), jnp.float32),
            pltpu.VMEM((_SZ,), jnp.float32),
            pltpu.VMEM((_OUT_PAD,), jnp.float32),
        ],
        compiler_params=pltpu.CompilerParams(needs_layout_passes=False),
    )
    flat = fn(outputs.reshape(-1), jnp.asarray(_ITAB), jnp.asarray(_FTAB))
    return flat.reshape(B, _C)


# speedup vs baseline: 1.7541x; 1.5389x over previous
import functools

import jax
import jax.numpy as jnp
import numpy as np
from jax import lax
from jax.experimental import pallas as pl
from jax.experimental.pallas import tpu as pltpu
from jax.experimental.pallas import tpu_sc as plsc

_C = 1000
_D = 10
_PAD = [max(16, 1 << d) for d in range(_D + 1)]
_MOFF = {}
_o = 0
for _d in range(1, _D + 1):
    _MOFF[_d] = _o
    _o += _PAD[_d]
_MSZ = _o
_POFF = {}
_o = 0
for _d in range(1, _D):
    _POFF[_d] = _o
    _o += _PAD[_d]
_PSZ = _o

_ORSTRIDE = 1008
_ODUMP = 2016
_OSZ = 3040


def _build_tables():
    g10 = np.zeros(_PAD[_D], np.int32)
    w10 = np.zeros(_PAD[_D], np.float32)
    sc10 = np.full(_PAD[_D], _ODUMP, np.int32)
    cnt = {d: np.zeros(_PAD[d], np.float64) for d in range(1, _D + 1)}
    pa = {d: np.ones(_PAD[d], np.float32) for d in range(1, _D + 1)}
    pb = {d: np.zeros(_PAD[d], np.float32) for d in range(1, _D + 1)}

    def rec(a, b, d, p):
        cnt[d][p] = b - a
        if b - a == 1:
            q = p
            for dd in range(d + 1, _D + 1):
                q = 2 * q
                cnt[dd][q] = 1
                pa[dd][q] = 0.0
                pb[dd][q] = 1.0
            g10[q] = a
            w10[q] = 1.0
            sc10[q] = a
        else:
            mid = a + (b - a) // 2
            rec(a, mid, d + 1, 2 * p)
            rec(mid, b, d + 1, 2 * p + 1)

    rec(0, _C // 2, 1, 0)
    rec(_C // 2, _C, 1, 1)

    we = {}
    wo = {}
    for d in range(1, _D):
        il = np.minimum(2 * np.arange(_PAD[d]), _PAD[d + 1] - 1)
        cl = cnt[d + 1][il]
        co = cnt[d + 1][np.minimum(il + 1, _PAD[d + 1] - 1)]
        cp = np.maximum(cnt[d], 1)
        we[d] = np.where(cnt[d] > 0, cl / cp, 0.0).astype(np.float32)
        wo[d] = np.where(cnt[d] > 0, co / cp, 0.0).astype(np.float32)

    ioffs = {}
    iparts = []

    def iadd(name, arr):
        ioffs[name] = sum(len(x) for x in iparts)
        iparts.append(arr.astype(np.int32))

    foffs = {}
    fparts = []

    def fadd(name, arr):
        foffs[name] = sum(len(x) for x in fparts)
        fparts.append(arr.astype(np.float32))

    iadd("g10", g10)
    iadd("sc10", sc10)
    fadd("w10", w10)
    for d in range(1, _D):
        fadd(("we", d), we[d])
        fadd(("wo", d), wo[d])
    for d in range(1, _D + 1):
        fadd(("pa", d), pa[d])
        fadd(("pb", d), pb[d])

    w10_need = [bool(np.any(w10[16 * k:16 * k + 16] != 1.0)) for k in range(_PAD[_D] // 16)]
    pa_need = {
        d: [bool(np.any(pa[d][16 * k:16 * k + 16] != 1.0)) for k in range(_PAD[d] // 16)]
        for d in range(1, _D + 1)
    }
    return (np.concatenate(iparts), np.concatenate(fparts), ioffs, foffs,
            w10_need, pa_need)


_ITAB, _FTAB, _IOFFS, _FOFFS, _W10_NEED, _PA_NEED = _build_tables()

_INFO = plsc.get_sparse_core_info()
_NW = _INFO.num_cores * _INFO.num_subcores


_TAKE_DN = lax.GatherDimensionNumbers(
    offset_dims=(), collapsed_slice_dims=(0,), start_index_map=(0,))


def _take(v, idx):
    return lax.gather(v, idx[:, None], _TAKE_DN, slice_sizes=(1,),
                      mode=lax.GatherScatterMode.PROMISE_IN_BOUNDS)


def _block_program(mb, ob, rbuf, obuf, mean_v, pp_v, itv, ftv, perms):
    ide, ido, sibp, plo, phi, masklo = perms

    def li(name, k):
        return itv[pl.ds(_IOFFS[name] + 16 * k, 16)]

    def lf(name, k):
        return ftv[pl.ds(_FOFFS[name] + 16 * k, 16)]

    for k in range(_PAD[_D] // 16):
        idx = li("g10", k)
        w = lf("w10", k) if _W10_NEED[k] else None
        for r in range(2):
            v = plsc.load_gather(rbuf, [idx + (mb + r * _C)])
            if w is not None:
                v = v * w
            mean_v[pl.ds(r * _MSZ + _MOFF[_D] + 16 * k, 16)] = v

    for d in range(_D - 1, 0, -1):
        for k in range(_PAD[d] // 16):
            we = lf(("we", d), k)
            wo = lf(("wo", d), k)
            for r in range(2):
                m0 = r * _MSZ
                if _PAD[d + 1] == 16:
                    c0 = mean_v[pl.ds(m0 + _MOFF[d + 1], 16)]
                    ev = _take(c0, ide)
                    od = _take(c0, ido)
                else:
                    c0 = mean_v[pl.ds(m0 + _MOFF[d + 1] + 32 * k, 16)]
                    c1 = mean_v[pl.ds(m0 + _MOFF[d + 1] + 32 * k + 16, 16)]
                    ev = jnp.where(masklo, _take(c0, ide), _take(c1, ide))
                    od = jnp.where(masklo, _take(c0, ido), _take(c1, ido))
                mean_v[pl.ds(m0 + _MOFF[d] + 16 * k, 16)] = ev * we + od * wo

    for d in range(1, _D + 1):
        for k in range(_PAD[d] // 16):
            need_pa = _PA_NEED[d][k]
            pav = lf(("pa", d), k) if need_pa else None
            pbv = lf(("pb", d), k) if need_pa else None
            sidx = li("sc10", k) if d == _D else None
            for r in range(2):
                m0 = r * _MSZ
                p0 = r * _PSZ
                s = mean_v[pl.ds(m0 + _MOFF[d] + 16 * k, 16)]
                sv = _take(s, sibp)
                p = 1.0 / (1.0 + jnp.exp(sv - s))
                if need_pa:
                    p = p * pav + pbv
                if d > 1:
                    parch = pp_v[pl.ds(p0 + _POFF[d - 1] + 16 * (k // 2), 16)]
                    p = p * _take(parch, plo if k % 2 == 0 else phi)
                if d < _D:
                    pp_v[pl.ds(p0 + _POFF[d] + 16 * k, 16)] = p
                else:
                    plsc.store_scatter(obuf, [sidx + (ob + r * _ORSTRIDE)], p)


def _sc_body(rows_per_w, x_hbm, itab_hbm, ftab_hbm, out_hbm,
             itv, ftv, rbuf, obuf, mean_v, pp_v, sin0, sin1, sout0, sout1):
    wid = lax.axis_index("s") * _INFO.num_cores + lax.axis_index("c")
    pltpu.sync_copy(itab_hbm, itv)
    pltpu.sync_copy(ftab_hbm, ftv)
    base = wid * rows_per_w * _C
    nblk = rows_per_w // 2

    lane = lax.iota(jnp.int32, 16)
    perms = ((2 * lane) & 15, ((2 * lane) & 15) + 1, lane ^ 1,
             lane >> 1, (lane >> 1) + 8, lane < 8)

    def in_copy(blk, roff, sem):
        pltpu.async_copy(x_hbm.at[pl.ds(base + blk * 2 * _C, 2 * _C)],
                         rbuf.at[pl.ds(roff, 2 * _C)], sem)

    def in_wait(sem):
        pltpu.make_async_copy(x_hbm.at[pl.ds(0, 2 * _C)],
                              rbuf.at[pl.ds(0, 2 * _C)], sem).wait()

    def out_copy(blk, ooff, sem):
        pltpu.async_copy(obuf.at[pl.ds(ooff, _C)],
                         out_hbm.at[pl.ds(base + blk * 2 * _C, _C)], sem)
        pltpu.async_copy(obuf.at[pl.ds(ooff + _ORSTRIDE, _C)],
                         out_hbm.at[pl.ds(base + blk * 2 * _C + _C, _C)], sem)

    def out_wait(sem):
        pltpu.make_async_copy(obuf.at[pl.ds(0, _C)],
                              out_hbm.at[pl.ds(0, _C)], sem).wait()
        pltpu.make_async_copy(obuf.at[pl.ds(0, _C)],
                              out_hbm.at[pl.ds(0, _C)], sem).wait()

    in_copy(0, 0, sin0)
    in_copy(1, 2 * _C, sin1)

    def body(j, carry):
        par = j & 1
        mb = par * (2 * _C)
        ob = par * _OSZ

        @pl.when(par == 0)
        def _():
            in_wait(sin0)

        @pl.when(par == 1)
        def _():
            in_wait(sin1)

        _block_program(mb, ob, rbuf, obuf, mean_v, pp_v, itv, ftv, perms)

        @pl.when(jnp.logical_and(j + 2 < nblk, par == 0))
        def _():
            in_copy(j + 2, mb, sin0)

        @pl.when(jnp.logical_and(j + 2 < nblk, par == 1))
        def _():
            in_copy(j + 2, mb, sin1)

        @pl.when(jnp.logical_and(j >= 2, par == 0))
        def _():
            out_wait(sout0)

        @pl.when(jnp.logical_and(j >= 2, par == 1))
        def _():
            out_wait(sout1)

        @pl.when(par == 0)
        def _():
            out_copy(j, ob, sout0)

        @pl.when(par == 1)
        def _():
            out_copy(j, ob, sout1)

        return carry

    lax.fori_loop(0, nblk, body, 0)
    out_wait(sout0)
    out_wait(sout1)


@jax.jit
def kernel(outputs):
    B = outputs.shape[0]
    assert B % (2 * _NW) == 0
    rows_per_w = B // _NW
    mesh = plsc.VectorSubcoreMesh(core_axis_name="c", subcore_axis_name="s")
    fn = pl.kernel(
        functools.partial(_sc_body, rows_per_w),
        out_type=jax.ShapeDtypeStruct((B * _C,), jnp.float32),
        mesh=mesh,
        scratch_types=[
            pltpu.VMEM((len(_ITAB),), jnp.int32),
            pltpu.VMEM((len(_FTAB),), jnp.float32),
            pltpu.VMEM((2 * 2 * _C,), jnp.float32),
            pltpu.VMEM((2 * _OSZ,), jnp.float32),
            pltpu.VMEM((2 * _MSZ,), jnp.float32),
            pltpu.VMEM((2 * _PSZ,), jnp.float32),
            pltpu.SemaphoreType.DMA,
            pltpu.SemaphoreType.DMA,
            pltpu.SemaphoreType.DMA,
            pltpu.SemaphoreType.DMA,
        ],
        compiler_params=pltpu.CompilerParams(needs_layout_passes=False),
    )
    flat = fn(outputs.reshape(-1), jnp.asarray(_ITAB), jnp.asarray(_FTAB))
    return flat.reshape(B, _C)


# speedup vs baseline: 4.0003x; 2.2805x over previous
import functools

import jax
import jax.numpy as jnp
import numpy as np
from jax import lax
from jax.experimental import pallas as pl
from jax.experimental.pallas import tpu as pltpu
from jax.experimental.pallas import tpu_sc as plsc

_C = 1000
_D = 10
_PAD = [max(16, 1 << d) for d in range(_D + 1)]
_MOFF = {}
_o = 0
for _d in range(1, _D + 1):
    _MOFF[_d] = _o
    _o += _PAD[_d]
_MSZ = _o
_POFF = {}
_o = 0
for _d in range(1, _D):
    _POFF[_d] = _o
    _o += _PAD[_d]
_PSZ = _o

_ORSTRIDE = 1008
_ODUMP = 2016
_OSZ = 3040


def _build_tables():
    g10 = np.zeros(_PAD[_D], np.int32)
    w10 = np.zeros(_PAD[_D], np.float32)
    sc10 = np.full(_PAD[_D], _ODUMP, np.int32)
    cnt = {d: np.zeros(_PAD[d], np.float64) for d in range(1, _D + 1)}
    pa = {d: np.ones(_PAD[d], np.float32) for d in range(1, _D + 1)}
    pb = {d: np.zeros(_PAD[d], np.float32) for d in range(1, _D + 1)}

    def rec(a, b, d, p):
        cnt[d][p] = b - a
        if b - a == 1:
            q = p
            for dd in range(d + 1, _D + 1):
                q = 2 * q
                cnt[dd][q] = 1
                pa[dd][q] = 0.0
                pb[dd][q] = 1.0
            g10[q] = a
            w10[q] = 1.0
            sc10[q] = a
        else:
            mid = a + (b - a) // 2
            rec(a, mid, d + 1, 2 * p)
            rec(mid, b, d + 1, 2 * p + 1)

    rec(0, _C // 2, 1, 0)
    rec(_C // 2, _C, 1, 1)

    we = {}
    wo = {}
    for d in range(1, _D):
        il = np.minimum(2 * np.arange(_PAD[d]), _PAD[d + 1] - 1)
        cl = cnt[d + 1][il]
        co = cnt[d + 1][np.minimum(il + 1, _PAD[d + 1] - 1)]
        cp = np.maximum(cnt[d], 1)
        we[d] = np.where(cnt[d] > 0, cl / cp, 0.0).astype(np.float32)
        wo[d] = np.where(cnt[d] > 0, co / cp, 0.0).astype(np.float32)

    ioffs = {}
    iparts = []

    def iadd(name, arr):
        ioffs[name] = sum(len(x) for x in iparts)
        iparts.append(arr.astype(np.int32))

    foffs = {}
    fparts = []

    def fadd(name, arr):
        foffs[name] = sum(len(x) for x in fparts)
        fparts.append(arr.astype(np.float32))

    iadd("g10", g10)
    iadd("sc10", sc10)
    fadd("w10", w10)
    for d in range(1, _D):
        fadd(("we", d), we[d])
        fadd(("wo", d), wo[d])
    for d in range(1, _D + 1):
        fadd(("pa", d), pa[d])
        fadd(("pb", d), pb[d])

    w10_need = [bool(np.any(w10[16 * k:16 * k + 16] != 1.0)) for k in range(_PAD[_D] // 16)]
    pa_need = {
        d: [bool(np.any(pa[d][16 * k:16 * k + 16] != 1.0)) for k in range(_PAD[d] // 16)]
        for d in range(1, _D + 1)
    }
    return (np.concatenate(iparts), np.concatenate(fparts), ioffs, foffs,
            w10_need, pa_need)


_ITAB, _FTAB, _IOFFS, _FOFFS, _W10_NEED, _PA_NEED = _build_tables()

_INFO = plsc.get_sparse_core_info()
_NW = _INFO.num_cores * _INFO.num_subcores


_TAKE_DN = lax.GatherDimensionNumbers(
    offset_dims=(), collapsed_slice_dims=(0,), start_index_map=(0,))


def _take(v, idx):
    return lax.gather(v, idx[:, None], _TAKE_DN, slice_sizes=(1,),
                      mode=lax.GatherScatterMode.PROMISE_IN_BOUNDS)


def _block_program(mb, ob, rbuf, obuf, mean_v, pp_v, itv, ftv, perms):
    ide, ido, sibp, plo, phi, masklo = perms

    def li(name, k):
        return itv[pl.ds(_IOFFS[name] + 16 * k, 16)]

    def lf(name, k):
        return ftv[pl.ds(_FOFFS[name] + 16 * k, 16)]

    def groups(nch, g):
        return [list(range(i, min(i + g, nch))) for i in range(0, nch, g)]

    rbase = [mb, mb + _C]
    for ks in groups(_PAD[_D] // 16, 4):
        idx = {k: li("g10", k) for k in ks}
        w = {k: lf("w10", k) for k in ks if _W10_NEED[k]}
        units = [(k, r) for k in ks for r in (0, 1)]
        v = {(k, r): plsc.load_gather(rbuf, [idx[k] + rbase[r]]) for k, r in units}
        v = {u: (v[u] * w[u[0]] if u[0] in w else v[u]) for u in units}
        for k, r in units:
            mean_v[pl.ds(r * _MSZ + _MOFF[_D] + 16 * k, 16)] = v[(k, r)]

    for d in range(_D - 1, 0, -1):
        for ks in groups(_PAD[d] // 16, 2):
            we = {k: lf(("we", d), k) for k in ks}
            wo = {k: lf(("wo", d), k) for k in ks}
            units = [(k, r) for k in ks for r in (0, 1)]
            c0 = {}
            c1 = {}
            for k, r in units:
                m0 = r * _MSZ
                if _PAD[d + 1] == 16:
                    c0[(k, r)] = mean_v[pl.ds(m0 + _MOFF[d + 1], 16)]
                else:
                    c0[(k, r)] = mean_v[pl.ds(m0 + _MOFF[d + 1] + 32 * k, 16)]
                    c1[(k, r)] = mean_v[pl.ds(m0 + _MOFF[d + 1] + 32 * k + 16, 16)]
            ev = {}
            od = {}
            for u in units:
                if u in c1:
                    ev[u] = jnp.where(masklo, _take(c0[u], ide), _take(c1[u], ide))
                    od[u] = jnp.where(masklo, _take(c0[u], ido), _take(c1[u], ido))
                else:
                    ev[u] = _take(c0[u], ide)
                    od[u] = _take(c0[u], ido)
            res = {u: ev[u] * we[u[0]] + od[u] * wo[u[0]] for u in units}
            for k, r in units:
                mean_v[pl.ds(r * _MSZ + _MOFF[d] + 16 * k, 16)] = res[(k, r)]

    for d in range(1, _D + 1):
        for ks in groups(_PAD[d] // 16, 4):
            units = [(k, r) for k in ks for r in (0, 1)]
            pav = {k: lf(("pa", d), k) for k in ks if _PA_NEED[d][k]}
            pbv = {k: lf(("pb", d), k) for k in ks if _PA_NEED[d][k]}
            sidx = {k: li("sc10", k) for k in ks} if d == _D else {}
            parch = {}
            if d > 1:
                for k, r in units:
                    key = (k // 2, r)
                    if key not in parch:
                        parch[key] = pp_v[
                            pl.ds(r * _PSZ + _POFF[d - 1] + 16 * (k // 2), 16)]
            s = {u: mean_v[pl.ds(u[1] * _MSZ + _MOFF[d] + 16 * u[0], 16)]
                 for u in units}
            sv = {u: _take(s[u], sibp) for u in units}
            e = {u: jnp.exp(sv[u] - s[u]) for u in units}
            p = {u: 1.0 / (1.0 + e[u]) for u in units}
            if pav:
                p = {u: (p[u] * pav[u[0]] + pbv[u[0]] if u[0] in pav else p[u])
                     for u in units}
            if d > 1:
                p = {u: p[u] * _take(parch[(u[0] // 2, u[1])],
                                     plo if u[0] % 2 == 0 else phi)
                     for u in units}
            for k, r in units:
                if d < _D:
                    pp_v[pl.ds(r * _PSZ + _POFF[d] + 16 * k, 16)] = p[(k, r)]
                else:
                    plsc.store_scatter(obuf, [sidx[k] + (ob + r * _ORSTRIDE)],
                                       p[(k, r)])


def _sc_body(rows_per_w, x_hbm, itab_hbm, ftab_hbm, out_hbm,
             itv, ftv, rbuf, obuf, mean_v, pp_v, sin0, sin1, sout0, sout1):
    wid = lax.axis_index("s") * _INFO.num_cores + lax.axis_index("c")
    pltpu.sync_copy(itab_hbm, itv)
    pltpu.sync_copy(ftab_hbm, ftv)
    base = wid * rows_per_w * _C
    nblk = rows_per_w // 2

    lane = lax.iota(jnp.int32, 16)
    perms = ((2 * lane) & 15, ((2 * lane) & 15) + 1, lane ^ 1,
             lane >> 1, (lane >> 1) + 8, lane < 8)

    def in_copy(blk, roff, sem):
        pltpu.async_copy(x_hbm.at[pl.ds(base + blk * 2 * _C, 2 * _C)],
                         rbuf.at[pl.ds(roff, 2 * _C)], sem)

    def in_wait(sem):
        pltpu.make_async_copy(x_hbm.at[pl.ds(0, 2 * _C)],
                              rbuf.at[pl.ds(0, 2 * _C)], sem).wait()

    def out_copy(blk, ooff, sem):
        pltpu.async_copy(obuf.at[pl.ds(ooff, _C)],
                         out_hbm.at[pl.ds(base + blk * 2 * _C, _C)], sem)
        pltpu.async_copy(obuf.at[pl.ds(ooff + _ORSTRIDE, _C)],
                         out_hbm.at[pl.ds(base + blk * 2 * _C + _C, _C)], sem)

    def out_wait(sem):
        pltpu.make_async_copy(obuf.at[pl.ds(0, _C)],
                              out_hbm.at[pl.ds(0, _C)], sem).wait()
        pltpu.make_async_copy(obuf.at[pl.ds(0, _C)],
                              out_hbm.at[pl.ds(0, _C)], sem).wait()

    in_copy(0, 0, sin0)
    in_copy(1, 2 * _C, sin1)

    def body(j, carry):
        par = j & 1
        mb = par * (2 * _C)
        ob = par * _OSZ

        @pl.when(par == 0)
        def _():
            in_wait(sin0)

        @pl.when(par == 1)
        def _():
            in_wait(sin1)

        _block_program(mb, ob, rbuf, obuf, mean_v, pp_v, itv, ftv, perms)

        @pl.when(jnp.logical_and(j + 2 < nblk, par == 0))
        def _():
            in_copy(j + 2, mb, sin0)

        @pl.when(jnp.logical_and(j + 2 < nblk, par == 1))
        def _():
            in_copy(j + 2, mb, sin1)

        @pl.when(jnp.logical_and(j >= 2, par == 0))
        def _():
            out_wait(sout0)

        @pl.when(jnp.logical_and(j >= 2, par == 1))
        def _():
            out_wait(sout1)

        @pl.when(par == 0)
        def _():
            out_copy(j, ob, sout0)

        @pl.when(par == 1)
        def _():
            out_copy(j, ob, sout1)

        return carry

    lax.fori_loop(0, nblk, body, 0)
    out_wait(sout0)
    out_wait(sout1)


@jax.jit
def kernel(outputs):
    B = outputs.shape[0]
    assert B % (2 * _NW) == 0
    rows_per_w = B // _NW
    mesh = plsc.VectorSubcoreMesh(core_axis_name="c", subcore_axis_name="s")
    fn = pl.kernel(
        functools.partial(_sc_body, rows_per_w),
        out_type=jax.ShapeDtypeStruct((B * _C,), jnp.float32),
        mesh=mesh,
        scratch_types=[
            pltpu.VMEM((len(_ITAB),), jnp.int32),
            pltpu.VMEM((len(_FTAB),), jnp.float32),
            pltpu.VMEM((2 * 2 * _C,), jnp.float32),
            pltpu.VMEM((2 * _OSZ,), jnp.float32),
            pltpu.VMEM((2 * _MSZ,), jnp.float32),
            pltpu.VMEM((2 * _PSZ,), jnp.float32),
            pltpu.SemaphoreType.DMA,
            pltpu.SemaphoreType.DMA,
            pltpu.SemaphoreType.DMA,
            pltpu.SemaphoreType.DMA,
        ],
        compiler_params=pltpu.CompilerParams(needs_layout_passes=False),
    )
    flat = fn(outputs.reshape(-1), jnp.asarray(_ITAB), jnp.asarray(_FTAB))
    return flat.reshape(B, _C)
